# baseline (speedup 1.0000x reference)
import functools
import math

import jax
import jax.numpy as jnp
from jax import lax
from jax.experimental import pallas as pl
from jax.experimental.pallas import tpu as pltpu
from jax.experimental.pallas import tpu_sc as plsc

S = 2048
H = 768
E = 8
F = 2048
NMOD = 3
TB = 128
NBLK = S // TB + E
NPOS = NBLK * TB
ENT_TH = math.log(4)

NC = 2
NS = 16
NW = NC * NS
TOK_W = S // NW


def _gate_body(x_ref, gw_ref, gb_ref, mm_ref, pos_ref, be_ref, aux_ref):
    x = x_ref[...]
    logits = jnp.dot(x, gw_ref[...], preferred_element_type=jnp.float32) + gb_ref[...]
    mx = jnp.max(logits, axis=-1, keepdims=True)
    ex = jnp.exp(logits - mx)
    p = ex / jnp.sum(ex, axis=-1, keepdims=True)

    iota_e = lax.broadcasted_iota(jnp.int32, (1, E), 1)
    is_max = logits >= jnp.max(logits, axis=-1, keepdims=True)
    e_id = jnp.min(jnp.where(is_max, iota_e, E), axis=-1, keepdims=True)

    ent = -jnp.sum(p * jnp.log(p + 1e-10), axis=-1)
    total = jnp.float32(0.0)
    nvalid = jnp.float32(0.0)
    for m_i in range(NMOD):
        mm = (mm_ref[:, m_i] > 0).astype(jnp.float32)
        cnt = jnp.sum(mm)
        denom = jnp.maximum(cnt, 1.0)
        local = jnp.sum(ent * mm) / denom
        marg = jnp.sum(p * mm[:, None], axis=0) / denom
        ge = -jnp.sum(marg * jnp.log(marg + 1e-10))
        aux = local + jax.nn.relu(ENT_TH + ge)
        valid = (cnt > 0).astype(jnp.float32)
        total = total + valid * aux
        nvalid = nvalid + valid
    aux_ref[...] = (total / jnp.maximum(nvalid, 1.0)) * jnp.ones((1, 1), jnp.float32)

    oh = (e_id == iota_e).astype(jnp.float32)
    c = oh
    k = 1
    while k < S:
        c = c + jnp.concatenate([jnp.zeros((k, E), jnp.float32), c[: S - k]], axis=0)
        k *= 2
    counts = c[S - 1 : S].astype(jnp.int32)
    nblk = (counts + TB - 1) // TB
    kk = lax.broadcasted_iota(jnp.int32, (E, E), 0)
    jj = lax.broadcasted_iota(jnp.int32, (E, E), 1)
    lt = (kk < jj).astype(jnp.int32)
    start_blk = jnp.sum(nblk.reshape(E, 1) * lt, axis=0, keepdims=True)
    group_start = (start_blk * TB).astype(jnp.float32)

    rank = jnp.sum(oh * c, axis=-1) - 1.0
    gs_t = jnp.sum(oh * group_start, axis=-1)
    pos_ref[...] = (gs_t + rank).astype(jnp.int32)

    ii = lax.broadcasted_iota(jnp.int32, (NBLK, 1), 0)
    be = jnp.sum((ii >= start_blk).astype(jnp.int32), axis=-1) - 1
    be_ref[...] = be


def _run_gate(xf, gate_w, gate_b, mm):
    return pl.pallas_call(
        _gate_body,
        out_shape=[
            jax.ShapeDtypeStruct((S,), jnp.int32),
            jax.ShapeDtypeStruct((NBLK,), jnp.int32),
            jax.ShapeDtypeStruct((1, 1), jnp.float32),
        ],
    )(xf, gate_w, gate_b.reshape(1, E), mm)


def _ffn_body(be_ref, pos_ref, x_ref, w1_ref, b1_ref, w2_ref, b2_ref, out_ref):
    del be_ref
    i = pl.program_id(0)
    slot = i * TB + lax.broadcasted_iota(jnp.int32, (TB, S), 0)
    p_i = (pos_ref[...] == slot).astype(jnp.float32)
    xb = jnp.dot(p_i, x_ref[...], preferred_element_type=jnp.float32)
    h = jnp.dot(xb, w1_ref[0], preferred_element_type=jnp.float32)
    h = h + b1_ref[0]
    h = 0.5 * h * (1.0 + lax.erf(h * (1.0 / math.sqrt(2.0))))
    o = jnp.dot(h, w2_ref[0], preferred_element_type=jnp.float32)
    out_ref[...] = o + b2_ref[0]


def _run_ffn(be, pos, x, w1, b1, w2, b2, interpret=False):
    return pl.pallas_call(
        _ffn_body,
        grid_spec=pltpu.PrefetchScalarGridSpec(
            num_scalar_prefetch=1,
            grid=(NBLK,),
            in_specs=[
                pl.BlockSpec((1, S), lambda i, be: (0, 0)),
                pl.BlockSpec((S, H), lambda i, be: (0, 0)),
                pl.BlockSpec((1, H, F), lambda i, be: (be[i], 0, 0)),
                pl.BlockSpec((1, 1, F), lambda i, be: (be[i], 0, 0)),
                pl.BlockSpec((1, F, H), lambda i, be: (be[i], 0, 0)),
                pl.BlockSpec((1, 1, H), lambda i, be: (be[i], 0, 0)),
            ],
            out_specs=pl.BlockSpec((TB, H), lambda i, be: (i, 0)),
        ),
        out_shape=jax.ShapeDtypeStruct((NPOS, H), jnp.float32),
        interpret=interpret,
    )(be, pos, x, w1, b1, w2, b2)


@functools.cache
def _make_sc_gather():
    mesh = plsc.VectorSubcoreMesh(core_axis_name="c", subcore_axis_name="s")

    @functools.partial(
        pl.kernel,
        mesh=mesh,
        out_type=jax.ShapeDtypeStruct((S, H), jnp.float32),
        scratch_types=[
            pltpu.VMEM((TOK_W,), jnp.int32),
            pltpu.VMEM((TOK_W, H), jnp.float32),
            pltpu.SemaphoreType.DMA,
        ],
    )
    def sc_gather(ys_hbm, pos_hbm, out_hbm, idx_v, rows_v, sem):
        wid = lax.axis_index("s") * NC + lax.axis_index("c")
        base = wid * TOK_W
        pltpu.sync_copy(pos_hbm.at[pl.ds(base, TOK_W)], idx_v)
        pltpu.async_copy(ys_hbm.at[idx_v], rows_v, sem).wait()
        pltpu.sync_copy(rows_v, out_hbm.at[pl.ds(base, TOK_W)])

    return sc_gather


def _sc_gather(ys_f32, pos):
    return _make_sc_gather()(ys_f32, pos)


def kernel(x, modality_mask, gate_w, gate_b, w1, b1, w2, b2):
    xf = x.reshape(S, H)
    mm = modality_mask.reshape(S, NMOD)
    pos, be, aux = _run_gate(xf, gate_w, gate_b, mm)

    ys = _run_ffn(be, pos.reshape(1, S), xf, w1, b1.reshape(E, 1, F), w2, b2.reshape(E, 1, H))
    out = _sc_gather(ys, pos)
    return out.astype(jnp.bfloat16).reshape(1, S, H), aux.reshape(())


# speedup vs baseline: 1.0683x; 1.0683x over previous
import functools
import math

import jax
import jax.numpy as jnp
from jax import lax
from jax.experimental import pallas as pl
from jax.experimental.pallas import tpu as pltpu
from jax.experimental.pallas import tpu_sc as plsc

S = 2048
H = 768
E = 8
F = 2048
NMOD = 3
TB = 128
NBLK = S // TB + E
NPOS = NBLK * TB
ENT_TH = math.log(4)

NC = 2
NS = 16
NW = NC * NS
TOK_W = S // NW


def _gate_body(x_ref, gw_ref, gb_ref, mm_ref, pos_ref, be_ref, aux_ref):
    x = x_ref[...]
    logits = jnp.dot(x, gw_ref[...], preferred_element_type=jnp.float32) + gb_ref[...]
    mx = jnp.max(logits, axis=-1, keepdims=True)
    ex = jnp.exp(logits - mx)
    p = ex / jnp.sum(ex, axis=-1, keepdims=True)

    iota_e = lax.broadcasted_iota(jnp.int32, (1, E), 1)
    is_max = logits >= jnp.max(logits, axis=-1, keepdims=True)
    e_id = jnp.min(jnp.where(is_max, iota_e, E), axis=-1, keepdims=True)

    ent = -jnp.sum(p * jnp.log(p + 1e-10), axis=-1)
    total = jnp.float32(0.0)
    nvalid = jnp.float32(0.0)
    for m_i in range(NMOD):
        mm = (mm_ref[:, m_i] > 0).astype(jnp.float32)
        cnt = jnp.sum(mm)
        denom = jnp.maximum(cnt, 1.0)
        local = jnp.sum(ent * mm) / denom
        marg = jnp.sum(p * mm[:, None], axis=0) / denom
        ge = -jnp.sum(marg * jnp.log(marg + 1e-10))
        aux = local + jax.nn.relu(ENT_TH + ge)
        valid = (cnt > 0).astype(jnp.float32)
        total = total + valid * aux
        nvalid = nvalid + valid
    aux_ref[...] = (total / jnp.maximum(nvalid, 1.0)) * jnp.ones((1, 1), jnp.float32)

    oh = (e_id == iota_e).astype(jnp.float32)
    c = oh
    k = 1
    while k < S:
        c = c + jnp.concatenate([jnp.zeros((k, E), jnp.float32), c[: S - k]], axis=0)
        k *= 2
    counts = c[S - 1 : S].astype(jnp.int32)
    nblk = (counts + TB - 1) // TB
    kk = lax.broadcasted_iota(jnp.int32, (E, E), 0)
    jj = lax.broadcasted_iota(jnp.int32, (E, E), 1)
    lt = (kk < jj).astype(jnp.int32)
    start_blk = jnp.sum(nblk.reshape(E, 1) * lt, axis=0, keepdims=True)
    group_start = (start_blk * TB).astype(jnp.float32)

    rank = jnp.sum(oh * c, axis=-1) - 1.0
    gs_t = jnp.sum(oh * group_start, axis=-1)
    pos_ref[...] = (gs_t + rank).astype(jnp.int32)

    ii = lax.broadcasted_iota(jnp.int32, (NBLK, 1), 0)
    be = jnp.sum((ii >= start_blk).astype(jnp.int32), axis=-1) - 1
    be_ref[...] = be


def _run_gate(xf, gate_w, gate_b, mm):
    return pl.pallas_call(
        _gate_body,
        out_shape=[
            jax.ShapeDtypeStruct((S,), jnp.int32),
            jax.ShapeDtypeStruct((NBLK,), jnp.int32),
            jax.ShapeDtypeStruct((1, 1), jnp.float32),
        ],
    )(xf, gate_w, gate_b.reshape(1, E), mm)


def _ffn_body(be_ref, xs_ref, w1_ref, b1_ref, w2_ref, b2_ref, out_ref):
    del be_ref
    xb = xs_ref[...]
    h = jnp.dot(xb, w1_ref[0], preferred_element_type=jnp.float32)
    h = h + b1_ref[0]
    h = 0.5 * h * (1.0 + lax.erf(h * (1.0 / math.sqrt(2.0))))
    o = jnp.dot(h, w2_ref[0], preferred_element_type=jnp.float32)
    out_ref[...] = o + b2_ref[0]


def _run_ffn(be, xs, w1, b1, w2, b2, interpret=False):
    return pl.pallas_call(
        _ffn_body,
        grid_spec=pltpu.PrefetchScalarGridSpec(
            num_scalar_prefetch=1,
            grid=(NBLK,),
            in_specs=[
                pl.BlockSpec((TB, H), lambda i, be: (i, 0)),
                pl.BlockSpec((1, H, F), lambda i, be: (be[i], 0, 0)),
                pl.BlockSpec((1, 1, F), lambda i, be: (be[i], 0, 0)),
                pl.BlockSpec((1, F, H), lambda i, be: (be[i], 0, 0)),
                pl.BlockSpec((1, 1, H), lambda i, be: (be[i], 0, 0)),
            ],
            out_specs=pl.BlockSpec((TB, H), lambda i, be: (i, 0)),
        ),
        out_shape=jax.ShapeDtypeStruct((NPOS, H), jnp.float32),
        interpret=interpret,
    )(be, xs, w1, b1, w2, b2)


@functools.cache
def _make_sc_kernels():
    mesh = plsc.VectorSubcoreMesh(core_axis_name="c", subcore_axis_name="s")

    @functools.partial(
        pl.kernel,
        mesh=mesh,
        out_type=jax.ShapeDtypeStruct((NPOS, H), jnp.float32),
        scratch_types=[
            pltpu.VMEM((TOK_W,), jnp.int32),
            pltpu.VMEM((TOK_W, H), jnp.float32),
            pltpu.SemaphoreType.DMA,
        ],
    )
    def sc_scatter(x_hbm, pos_hbm, out_hbm, idx_v, rows_v, sem):
        wid = lax.axis_index("s") * NC + lax.axis_index("c")
        base = wid * TOK_W
        pltpu.sync_copy(pos_hbm.at[pl.ds(base, TOK_W)], idx_v)
        pltpu.sync_copy(x_hbm.at[pl.ds(base, TOK_W)], rows_v)
        pltpu.async_copy(rows_v, out_hbm.at[idx_v], sem).wait()

    @functools.partial(
        pl.kernel,
        mesh=mesh,
        out_type=jax.ShapeDtypeStruct((S, H), jnp.float32),
        scratch_types=[
            pltpu.VMEM((TOK_W,), jnp.int32),
            pltpu.VMEM((TOK_W, H), jnp.float32),
            pltpu.SemaphoreType.DMA,
        ],
    )
    def sc_gather(ys_hbm, pos_hbm, out_hbm, idx_v, rows_v, sem):
        wid = lax.axis_index("s") * NC + lax.axis_index("c")
        base = wid * TOK_W
        pltpu.sync_copy(pos_hbm.at[pl.ds(base, TOK_W)], idx_v)
        pltpu.async_copy(ys_hbm.at[idx_v], rows_v, sem).wait()
        pltpu.sync_copy(rows_v, out_hbm.at[pl.ds(base, TOK_W)])

    return sc_scatter, sc_gather


def _sc_scatter(x_f32, pos):
    return _make_sc_kernels()[0](x_f32, pos)


def _sc_gather(ys_f32, pos):
    return _make_sc_kernels()[1](ys_f32, pos)


def kernel(x, modality_mask, gate_w, gate_b, w1, b1, w2, b2):
    xf = x.reshape(S, H)
    mm = modality_mask.reshape(S, NMOD)
    pos, be, aux = _run_gate(xf, gate_w, gate_b, mm)

    xs = _sc_scatter(xf, pos)
    ys = _run_ffn(be, xs, w1, b1.reshape(E, 1, F), w2, b2.reshape(E, 1, H))
    out = _sc_gather(ys, pos)
    return out.astype(jnp.bfloat16).reshape(1, S, H), aux.reshape(())
